# baseline (speedup 1.0000x reference)
import jax
import jax.numpy as jnp
from jax.experimental import pallas as pl
from jax.experimental.shard_map import shard_map
from jax.sharding import Mesh, PartitionSpec as P

D_MODEL = 3072
D_SPARSE = 24576
K_SPARSE = 128
BATCH = 128

_BN = 1024
_BR = 32


def _matmul_kernel(h_ref, w_ref, pb_ref, eb_ref, out_ref):
    x = h_ref[...] - pb_ref[...]
    acts = jax.lax.dot_general(
        x, w_ref[...],
        dimension_numbers=(((1,), (1,)), ((), ())),
        preferred_element_type=jnp.float32,
    ) + eb_ref[...]
    s = jax.lax.bitcast_convert_type(acts, jnp.int32)
    out_ref[...] = jnp.where(s >= 0, s, s ^ jnp.int32(0x7FFFFFFF))


def _tm_kernel(y_ref, t_ref, m_ref):
    y = y_ref[...]
    rows = y.shape[0]
    k = jnp.int32(K_SPARSE)

    yl = y.reshape(rows, D_SPARSE // 128, 128)
    lane_max = jnp.max(yl, axis=1)
    lo0 = jnp.min(lane_max, axis=1, keepdims=True)
    hi0 = jnp.max(lane_max, axis=1, keepdims=True) + 1
    cnt0 = jnp.sum((y >= lo0).astype(jnp.int32), axis=1, keepdims=True)

    def _active(lo, hi, cnt):
        d = jax.lax.bitcast_convert_type(hi - lo, jnp.uint32)
        return (cnt != k) & (d > jnp.uint32(1))

    def cond(state):
        lo, hi, cnt = state
        return jnp.any(_active(lo, hi, cnt))

    def body(state):
        lo, hi, cnt = state
        act = _active(lo, hi, cnt)
        mid = (lo & hi) + ((lo ^ hi) >> 1)
        c = jnp.sum((y >= mid).astype(jnp.int32), axis=1, keepdims=True)
        ge = c >= k
        lo = jnp.where(act & ge, mid, lo)
        cnt = jnp.where(act & ge, c, cnt)
        hi = jnp.where(act & (~ge), mid, hi)
        return lo, hi, cnt

    t, _, cnt_ge = jax.lax.while_loop(cond, body, (lo0, hi0, cnt0))

    t_ref[...] = jnp.broadcast_to(t, (rows, 128))
    m_ref[...] = jnp.full((rows, 128), jnp.int32(D_SPARSE - 1))

    @pl.when(jnp.logical_not(jnp.all(cnt_ge == k)))
    def _():
        cnt_gt = jnp.sum((y > t).astype(jnp.int32), axis=1, keepdims=True)
        extras = k - cnt_gt
        idx = jax.lax.broadcasted_iota(jnp.int32, y.shape, 1)
        tie = y == t

        def ibody(i, m):
            b = 14 - i
            c = m + (jnp.int32(1) << b)
            cnt = jnp.sum((tie & (idx <= c)).astype(jnp.int32), axis=1,
                          keepdims=True)
            return jnp.where(cnt < extras, c, m)

        m0 = jnp.full((rows, 1), jnp.int32(-1))
        m = jax.lax.fori_loop(0, 15, ibody, m0)
        mm = jnp.where(cnt_ge == k, jnp.int32(D_SPARSE - 1), m + 1)
        m_ref[...] = jnp.broadcast_to(mm, (rows, 128))


def _mask_kernel(y_ref, t_ref, m_ref, out_ref):
    y = y_ref[...]
    t = t_ref[...][:, 0:1]
    m = m_ref[...][:, 0:1]
    idx = jax.lax.broadcasted_iota(jnp.int32, y.shape, 1)
    keep = ((y > t) | ((y == t) & (idx <= m))) & (y > 0)
    out_ref[...] = jnp.where(
        keep, jax.lax.bitcast_convert_type(y, jnp.float32), 0.0)


def _make_sharded(n):
    half = D_SPARSE // n
    rows_per = BATCH // n
    brs = min(_BR, rows_per)

    def _shard_fn(h, w, pb, eb):
        keys_half = pl.pallas_call(
            _matmul_kernel,
            grid=(half // _BN,),
            in_specs=[
                pl.BlockSpec((BATCH, D_MODEL), lambda i: (0, 0)),
                pl.BlockSpec((_BN, D_MODEL), lambda i: (i, 0)),
                pl.BlockSpec((1, D_MODEL), lambda i: (0, 0)),
                pl.BlockSpec((1, _BN), lambda i: (0, i)),
            ],
            out_specs=pl.BlockSpec((BATCH, _BN), lambda i: (0, i)),
            out_shape=jax.ShapeDtypeStruct((BATCH, half), jnp.int32),
        )(h, w, pb, eb)

        keys_full = jax.lax.all_gather(keys_half, "x", axis=1, tiled=True)
        didx = jax.lax.axis_index("x")
        my_rows = jax.lax.dynamic_slice_in_dim(
            keys_full, didx * rows_per, rows_per, 0)

        t_loc, m_loc = pl.pallas_call(
            _tm_kernel,
            grid=(rows_per // brs,),
            in_specs=[pl.BlockSpec((brs, D_SPARSE), lambda i: (i, 0))],
            out_specs=[pl.BlockSpec((brs, 128), lambda i: (i, 0)),
                       pl.BlockSpec((brs, 128), lambda i: (i, 0))],
            out_shape=[jax.ShapeDtypeStruct((rows_per, 128), jnp.int32),
                       jax.ShapeDtypeStruct((rows_per, 128), jnp.int32)],
        )(my_rows)

        t_all = jax.lax.all_gather(t_loc, "x", axis=0, tiled=True)
        m_all = jax.lax.all_gather(m_loc, "x", axis=0, tiled=True)
        m_all = m_all - didx * half

        out_half = pl.pallas_call(
            _mask_kernel,
            grid=(BATCH // _BR,),
            in_specs=[
                pl.BlockSpec((_BR, half), lambda i: (i, 0)),
                pl.BlockSpec((_BR, 128), lambda i: (i, 0)),
                pl.BlockSpec((_BR, 128), lambda i: (i, 0)),
            ],
            out_specs=pl.BlockSpec((_BR, half), lambda i: (i, 0)),
            out_shape=jax.ShapeDtypeStruct((BATCH, half), jnp.float32),
        )(keys_half, t_all, m_all)
        return out_half

    return _shard_fn


def kernel(h, W_enc, pre_bias, enc_bias):
    n_avail = jax.device_count()
    n = 2 if n_avail >= 2 else 1
    mesh = Mesh(jax.devices()[:n], ("x",))

    pb = pre_bias.reshape(1, D_MODEL)
    eb = enc_bias.reshape(1, D_SPARSE)

    fn = shard_map(
        _make_sharded(n),
        mesh=mesh,
        in_specs=(P(), P("x", None), P(), P(None, "x")),
        out_specs=P(None, "x"),
        check_rep=False,
    )
    return fn(h, W_enc, pb, eb)


# speedup vs baseline: 1.0787x; 1.0787x over previous
import jax
import jax.numpy as jnp
from jax.experimental import pallas as pl
from jax.experimental.shard_map import shard_map
from jax.sharding import Mesh, PartitionSpec as P

D_MODEL = 3072
D_SPARSE = 24576
K_SPARSE = 128
BATCH = 128

_BN = 1024
_BR = 32


def _matmul_kernel(h_ref, w_ref, pb_ref, eb_ref, out_ref):
    x = h_ref[...] - pb_ref[...]
    acts = jax.lax.dot_general(
        x, w_ref[...],
        dimension_numbers=(((1,), (1,)), ((), ())),
        preferred_element_type=jnp.float32,
    ) + eb_ref[...]
    s = jax.lax.bitcast_convert_type(acts, jnp.int32)
    out_ref[...] = jnp.where(s >= 0, s, s ^ jnp.int32(0x7FFFFFFF))


def _tm_kernel(y_ref, t_ref, m_ref):
    y = y_ref[...]
    rows = y.shape[0]
    k = jnp.int32(K_SPARSE)

    yl = y.reshape(rows, D_SPARSE // 128, 128)
    lane_max = jnp.max(yl, axis=1)
    lo0 = jnp.min(lane_max, axis=1, keepdims=True)
    hi0 = jnp.max(lane_max, axis=1, keepdims=True) + 1
    cnt0 = jnp.sum((y >= lo0).astype(jnp.int32), axis=1, keepdims=True)

    def _active(lo, hi, cnt):
        d = jax.lax.bitcast_convert_type(hi - lo, jnp.uint32)
        return (cnt != k) & (d > jnp.uint32(1))

    def cond(state):
        lo, hi, cnt = state
        return jnp.any(_active(lo, hi, cnt))

    def body(state):
        lo, hi, cnt = state
        act = _active(lo, hi, cnt)
        mid = (lo & hi) + ((lo ^ hi) >> 1)
        c = jnp.sum((y >= mid).astype(jnp.int32), axis=1, keepdims=True)
        ge = c >= k
        lo = jnp.where(act & ge, mid, lo)
        cnt = jnp.where(act & ge, c, cnt)
        hi = jnp.where(act & (~ge), mid, hi)
        return lo, hi, cnt

    t, _, cnt_ge = jax.lax.while_loop(cond, body, (lo0, hi0, cnt0))

    t_ref[...] = jnp.broadcast_to(t, (rows, 128))
    m_ref[...] = jnp.full((rows, 128), jnp.int32(D_SPARSE - 1))

    @pl.when(jnp.logical_not(jnp.all(cnt_ge == k)))
    def _():
        cnt_gt = jnp.sum((y > t).astype(jnp.int32), axis=1, keepdims=True)
        extras = k - cnt_gt
        idx = jax.lax.broadcasted_iota(jnp.int32, y.shape, 1)
        tie = y == t

        def ibody(i, m):
            b = 14 - i
            c = m + (jnp.int32(1) << b)
            cnt = jnp.sum((tie & (idx <= c)).astype(jnp.int32), axis=1,
                          keepdims=True)
            return jnp.where(cnt < extras, c, m)

        m0 = jnp.full((rows, 1), jnp.int32(-1))
        m = jax.lax.fori_loop(0, 15, ibody, m0)
        mm = jnp.where(cnt_ge == k, jnp.int32(D_SPARSE - 1), m + 1)
        m_ref[...] = jnp.broadcast_to(mm, (rows, 128))


def _mask_kernel(y_ref, t_ref, m_ref, out_ref):
    y = y_ref[...]
    t = t_ref[...][:, 0:1]
    m = m_ref[...][:, 0:1]
    idx = jax.lax.broadcasted_iota(jnp.int32, y.shape, 1)
    keep = ((y > t) | ((y == t) & (idx <= m))) & (y > 0)
    out_ref[...] = jnp.where(
        keep, jax.lax.bitcast_convert_type(y, jnp.float32), 0.0)


def _make_sharded(n):
    half = D_SPARSE // n
    rows_per = BATCH // n
    brs = min(_BR, rows_per)

    def _shard_fn(h, w, pb, eb):
        keys_half = pl.pallas_call(
            _matmul_kernel,
            grid=(half // _BN,),
            in_specs=[
                pl.BlockSpec((BATCH, D_MODEL), lambda i: (0, 0)),
                pl.BlockSpec((_BN, D_MODEL), lambda i: (i, 0)),
                pl.BlockSpec((1, D_MODEL), lambda i: (0, 0)),
                pl.BlockSpec((1, _BN), lambda i: (0, i)),
            ],
            out_specs=pl.BlockSpec((BATCH, _BN), lambda i: (0, i)),
            out_shape=jax.ShapeDtypeStruct((BATCH, half), jnp.int32),
        )(h, w, pb, eb)

        keys_full = jax.lax.all_gather(keys_half, "x", axis=1, tiled=True)
        didx = jax.lax.axis_index("x")

        t_all, m_all = pl.pallas_call(
            _tm_kernel,
            grid=(BATCH // _BR,),
            in_specs=[pl.BlockSpec((_BR, D_SPARSE), lambda i: (i, 0))],
            out_specs=[pl.BlockSpec((_BR, 128), lambda i: (i, 0)),
                       pl.BlockSpec((_BR, 128), lambda i: (i, 0))],
            out_shape=[jax.ShapeDtypeStruct((BATCH, 128), jnp.int32),
                       jax.ShapeDtypeStruct((BATCH, 128), jnp.int32)],
        )(keys_full)

        m_all = m_all - didx * half

        out_half = pl.pallas_call(
            _mask_kernel,
            grid=(BATCH // _BR,),
            in_specs=[
                pl.BlockSpec((_BR, half), lambda i: (i, 0)),
                pl.BlockSpec((_BR, 128), lambda i: (i, 0)),
                pl.BlockSpec((_BR, 128), lambda i: (i, 0)),
            ],
            out_specs=pl.BlockSpec((_BR, half), lambda i: (i, 0)),
            out_shape=jax.ShapeDtypeStruct((BATCH, half), jnp.float32),
        )(keys_half, t_all, m_all)
        return out_half

    return _shard_fn


def kernel(h, W_enc, pre_bias, enc_bias):
    n_avail = jax.device_count()
    n = 2 if n_avail >= 2 else 1
    mesh = Mesh(jax.devices()[:n], ("x",))

    pb = pre_bias.reshape(1, D_MODEL)
    eb = enc_bias.reshape(1, D_SPARSE)

    fn = shard_map(
        _make_sharded(n),
        mesh=mesh,
        in_specs=(P(), P("x", None), P(), P(None, "x")),
        out_specs=P(None, "x"),
        check_rep=False,
    )
    return fn(h, W_enc, pb, eb)


# speedup vs baseline: 5.6479x; 5.2357x over previous
import jax
import jax.numpy as jnp
from jax.experimental import pallas as pl

D_MODEL = 3072
D_SPARSE = 24576
K_SPARSE = 128
BATCH = 128

_BN = 1024
_BR = 32


def _matmul_kernel(h_ref, w_ref, pb_ref, eb_ref, out_ref, lmax_ref):
    i = pl.program_id(0)
    x = h_ref[...] - pb_ref[...]
    acts = jax.lax.dot_general(
        x, w_ref[...],
        dimension_numbers=(((1,), (1,)), ((), ())),
        preferred_element_type=jnp.float32,
    ) + eb_ref[...]
    s = jax.lax.bitcast_convert_type(acts, jnp.int32)
    y = jnp.where(s >= 0, s, s ^ jnp.int32(0x7FFFFFFF))
    out_ref[...] = y

    lm = jnp.max(y.reshape(BATCH, _BN // 128, 128), axis=1)

    @pl.when(i == 0)
    def _():
        lmax_ref[...] = lm

    @pl.when(i > 0)
    def _():
        lmax_ref[...] = jnp.maximum(lmax_ref[...], lm)


def _select_kernel(y_ref, lmax_ref, out_ref):
    y = y_ref[...]
    rows = y.shape[0]
    k = jnp.int32(K_SPARSE)

    lane_max = lmax_ref[...]
    lo0 = jnp.min(lane_max, axis=1, keepdims=True)
    hi0 = jnp.max(lane_max, axis=1, keepdims=True) + 1
    cnt0 = jnp.sum((y >= lo0).astype(jnp.int32), axis=1, keepdims=True)

    def _active(lo, hi, cnt):
        d = jax.lax.bitcast_convert_type(hi - lo, jnp.uint32)
        return (cnt != k) & (d > jnp.uint32(1))

    def cond(state):
        lo, hi, cnt, _nhi, _it = state
        return jnp.any(_active(lo, hi, cnt))

    def body(state):
        lo, hi, cnt, nhi, it = state
        act = _active(lo, hi, cnt)
        width = (hi - lo).astype(jnp.float32)
        frac = (cnt - k).astype(jnp.float32) / (cnt - nhi).astype(jnp.float32)
        delta = (frac * width).astype(jnp.int32)
        mid_rf = lo + jnp.clip(delta, 1, hi - lo - 1)
        mid_bi = (lo & hi) + ((lo ^ hi) >> 1)
        mid = jnp.where(it % 3 == 2, mid_bi, mid_rf)
        c = jnp.sum((y >= mid).astype(jnp.int32), axis=1, keepdims=True)
        ge = c >= k
        lo = jnp.where(act & ge, mid, lo)
        cnt = jnp.where(act & ge, c, cnt)
        hi = jnp.where(act & (~ge), mid, hi)
        nhi = jnp.where(act & (~ge), c, nhi)
        return lo, hi, cnt, nhi, it + 1

    nhi0 = jnp.zeros((rows, 1), jnp.int32)
    t, _, cnt_ge, _, _ = jax.lax.while_loop(
        cond, body, (lo0, hi0, cnt0, nhi0, jnp.int32(0)))

    out_ref[...] = jnp.where(
        (y >= t) & (y > 0), jax.lax.bitcast_convert_type(y, jnp.float32),
        0.0)

    @pl.when(jnp.logical_not(jnp.all(cnt_ge == k)))
    def _():
        cnt_gt = jnp.sum((y > t).astype(jnp.int32), axis=1, keepdims=True)
        extras = k - cnt_gt
        idx = jax.lax.broadcasted_iota(jnp.int32, y.shape, 1)
        tie = y == t

        def ibody(i, m):
            b = 14 - i
            c = m + (jnp.int32(1) << b)
            cnt = jnp.sum((tie & (idx <= c)).astype(jnp.int32), axis=1,
                          keepdims=True)
            return jnp.where(cnt < extras, c, m)

        m0 = jnp.full((rows, 1), jnp.int32(-1))
        m = jax.lax.fori_loop(0, 15, ibody, m0)

        keep = ((y > t) | (tie & (idx <= m + 1))) & (y > 0)
        out_ref[...] = jnp.where(
            keep, jax.lax.bitcast_convert_type(y, jnp.float32), 0.0)


@jax.jit
def kernel(h, W_enc, pre_bias, enc_bias):
    pb = pre_bias.reshape(1, D_MODEL)
    eb = enc_bias.reshape(1, D_SPARSE)

    keys, lmax = pl.pallas_call(
        _matmul_kernel,
        grid=(D_SPARSE // _BN,),
        in_specs=[
            pl.BlockSpec((BATCH, D_MODEL), lambda i: (0, 0)),
            pl.BlockSpec((_BN, D_MODEL), lambda i: (i, 0)),
            pl.BlockSpec((1, D_MODEL), lambda i: (0, 0)),
            pl.BlockSpec((1, _BN), lambda i: (0, i)),
        ],
        out_specs=[pl.BlockSpec((BATCH, _BN), lambda i: (0, i)),
                   pl.BlockSpec((BATCH, 128), lambda i: (0, 0))],
        out_shape=[jax.ShapeDtypeStruct((BATCH, D_SPARSE), jnp.int32),
                   jax.ShapeDtypeStruct((BATCH, 128), jnp.int32)],
    )(h, W_enc, pb, eb)

    out = pl.pallas_call(
        _select_kernel,
        grid=(BATCH // _BR,),
        in_specs=[pl.BlockSpec((_BR, D_SPARSE), lambda i: (i, 0)),
                  pl.BlockSpec((_BR, 128), lambda i: (i, 0))],
        out_specs=pl.BlockSpec((_BR, D_SPARSE), lambda i: (i, 0)),
        out_shape=jax.ShapeDtypeStruct((BATCH, D_SPARSE), jnp.float32),
    )(keys, lmax)
    return out
